# baseline (speedup 1.0000x reference)
import dataclasses
import functools

import jax
import jax.numpy as jnp
import numpy as np
from jax import lax
from jax.experimental import pallas as pl
from jax.experimental.pallas import tpu as pltpu
from jax.experimental.pallas import tpu_sc as plsc

NUM_LEVELS = 16
F = 2
BASE_RES = 16
FINEST = 2048
T = 1 << 19
MASK = T - 1
PLS = float(np.exp2(np.log2(FINEST / BASE_RES) / (NUM_LEVELS - 1)))
P1_I32 = np.int32(np.uint32(2654435761).astype(np.int32))

N_PTS = 1048576
NC, NS = 2, 16
NW = NC * NS
PW = N_PTS // NW
C = 128
K = C // 128
NCHUNK = PW // C
G = N_PTS // C
HROWS = 40

_SCALES, _RES, _DENSE = [], [], []
for _l in range(NUM_LEVELS):
    _s = BASE_RES * (PLS ** _l) - 1.0
    _r = int(np.ceil(_s)) + 1
    _SCALES.append(np.float32(_s))
    _RES.append(_r)
    _DENSE.append(_r * _r <= T)
_DENSE_LVLS = [l for l in range(NUM_LEVELS) if _DENSE[l]]
_HASH_LVLS = [l for l in range(NUM_LEVELS) if not _DENSE[l]]
_DROWS = {l: _RES[l] * (_RES[l] + 1) + 2 for l in _DENSE_LVLS}

_SEG = {}
_seg = 0
for _l in range(NUM_LEVELS):
    _SEG[_l] = _seg
    _seg += K if _DENSE[_l] else 4 * K
NSEG = _seg
_NHSEG = 4 * K * len(_HASH_LVLS)


def _pack_dense(table, l):
    r = _RES[l]
    n = _DROWS[l]
    tl = table[l]
    return jnp.concatenate(
        [tl[0:n], tl[1:n + 1], tl[r:n + r], tl[r + 1:n + r + 1]], axis=-1)


def _sc_encode(xp, tables):
    mesh = plsc.VectorSubcoreMesh(core_axis_name="c", subcore_axis_name="s")
    cp = pltpu.CompilerParams()
    if "needs_layout_passes" in pltpu.CompilerParams.__dataclass_fields__:
        cp = dataclasses.replace(cp, needs_layout_passes=False)
    if "use_tc_tiling_on_sc" in pltpu.CompilerParams.__dataclass_fields__:
        cp = dataclasses.replace(cp, use_tc_tiling_on_sc=False)

    @functools.partial(
        pl.kernel,
        compiler_params=cp,
        out_type=jax.ShapeDtypeStruct((G, HROWS, C), jnp.float32),
        mesh=mesh,
        scratch_types=[
            pltpu.VMEM((C,), jnp.float32),
            pltpu.VMEM((C,), jnp.float32),
            pltpu.VMEM((NUM_LEVELS, C), jnp.float32),
            pltpu.VMEM((NUM_LEVELS, C), jnp.float32),
            pltpu.VMEM((NSEG, 128), jnp.int32),
            pltpu.VMEM((_NHSEG, 128), jnp.int32),
            pltpu.VMEM((NSEG * 128, 8), jnp.float32),
            pltpu.VMEM((HROWS, C), jnp.float32),
            pltpu.SemaphoreType.DMA,
            pltpu.SemaphoreType.DMA,
        ],
    )
    def enc_kernel(*refs):
        xp_hbm = refs[0]
        tbl_hbm = list(refs[1:1 + NUM_LEVELS])
        enc_hbm = refs[1 + NUM_LEVELS]
        (xv0, xv1, sfx, sfy, idxv, idxw, rowsv, outv,
         sem_g, sem_o) = refs[2 + NUM_LEVELS:]
        wid = lax.axis_index("s") * NC + lax.axis_index("c")
        io16 = lax.iota(jnp.int32, 16)

        @pl.loop(0, C, step=16)
        def _zpad(i):
            zf = jnp.zeros((16,), jnp.float32)
            for r in range(2 + 2 * NUM_LEVELS, HROWS):
                outv[r, pl.ds(i, 16)] = zf

        @pl.loop(0, NCHUNK)
        def _chunk(chunk):
            g = wid * NCHUNK + chunk
            pltpu.sync_copy(xp_hbm.at[2 * g], xv0)
            pltpu.sync_copy(xp_hbm.at[2 * g + 1], xv1)

            copies = []
            for l in range(NUM_LEVELS):
                scale = _SCALES[l]
                res = _RES[l]
                s0 = _SEG[l]
                for k in range(K):
                    @pl.loop(0, 128, step=16)
                    def _ixb(j, l=l, scale=scale, res=res, k=k, s0=s0):
                        i = k * 128 + j
                        vx = xv0[pl.ds(i, 16)]
                        vy = xv1[pl.ds(i, 16)]
                        px = vx * scale + 0.5
                        py = vy * scale + 0.5
                        pix = px.astype(jnp.int32)
                        piy = py.astype(jnp.int32)
                        fx = px - pix.astype(jnp.float32)
                        fy = py - piy.astype(jnp.float32)
                        sfx[l, pl.ds(i, 16)] = fx * fx * (3.0 - 2.0 * fx)
                        sfy[l, pl.ds(i, 16)] = fy * fy * (3.0 - 2.0 * fy)
                        if _DENSE[l]:
                            idxv[s0 + k, pl.ds(j, 16)] = piy * res + pix
                        else:
                            hy0 = piy * P1_I32
                            hy1 = hy0 + P1_I32
                            px1 = pix + 1
                            h0 = (_SEG[l] - _SEG[_HASH_LVLS[0]])
                            for c, gg in enumerate((
                                    (pix ^ hy0) & MASK, (px1 ^ hy0) & MASK,
                                    (pix ^ hy1) & MASK, (px1 ^ hy1) & MASK)):
                                idxv[s0 + c * K + k, pl.ds(j, 16)] = (
                                    lax.shift_right_logical(gg, 2))
                                idxw[h0 + c * K + k, pl.ds(j, 16)] = (
                                    (gg & 3) * 2)

                nseg_l = K if _DENSE[l] else 4 * K
                for s in range(s0, s0 + nseg_l):
                    copies.append(pltpu.async_copy(
                        tbl_hbm[l].at[idxv.at[s]],
                        rowsv.at[pl.ds(s * 128, 128)],
                        sem_g))

            for cp_ in copies:
                cp_.wait()

            @pl.when(chunk > 0)
            def _():
                pltpu.make_async_copy(
                    outv, enc_hbm.at[g], sem_o).wait()

            @pl.loop(0, C, step=16)
            def _xrow(i):
                outv[0, pl.ds(i, 16)] = 2.0 * xv0[pl.ds(i, 16)] - 1.0
                outv[1, pl.ds(i, 16)] = 2.0 * xv1[pl.ds(i, 16)] - 1.0

            z16 = jnp.zeros((16,), jnp.int32)
            wcol = [z16 + w for w in range(8)]
            for l in range(NUM_LEVELS):
                s0 = _SEG[l]

                @pl.loop(0, C, step=16)
                def _cmb(i, l=l, s0=s0):
                    sx = sfx[l, pl.ds(i, 16)]
                    sy = sfy[l, pl.ds(i, 16)]
                    wx0 = 1.0 - sx
                    wy0 = 1.0 - sy
                    w00 = wx0 * wy0
                    w10 = sx * wy0
                    w01 = wx0 * sy
                    w11 = sx * sy
                    pi = io16 + i
                    if _DENSE[l]:
                        row = pi + s0 * 128
                        gv = [plsc.load_gather(rowsv, [row, wcol[w]])
                              for w in range(8)]
                        a0 = (w00 * gv[0] + w10 * gv[2]
                              + w01 * gv[4] + w11 * gv[6])
                        a1 = (w00 * gv[1] + w10 * gv[3]
                              + w01 * gv[5] + w11 * gv[7])
                    else:
                        h0 = s0 - _SEG[_HASH_LVLS[0]]
                        ws = [w00, w10, w01, w11]
                        a0 = None
                        a1 = None
                        for c in range(4):
                            r = pi + (s0 + c * K) * 128
                            wb = idxw[h0 + c * K, pl.ds(i, 16)]
                            f0 = plsc.load_gather(rowsv, [r, wb])
                            f1 = plsc.load_gather(rowsv, [r, wb + 1])
                            a0 = ws[c] * f0 if a0 is None else a0 + ws[c] * f0
                            a1 = ws[c] * f1 if a1 is None else a1 + ws[c] * f1
                    outv[2 + 2 * l, pl.ds(i, 16)] = a0
                    outv[3 + 2 * l, pl.ds(i, 16)] = a1

            pltpu.async_copy(outv, enc_hbm.at[g], sem_o)

        pltpu.make_async_copy(outv, enc_hbm.at[0], sem_o).wait()

    return enc_kernel(xp, *tables)


def _mlp_kernel(h_ref, w1tp_ref, w2_ref, w3_ref, out_ref):
    h3 = h_ref[...]
    gb = h3.shape[0]
    a1 = jax.lax.dot_general(h3, w1tp_ref[...], (((1,), (1,)), ((), ())),
                             preferred_element_type=jnp.float32)
    a1 = jnp.maximum(a1, 0.0).reshape(gb * C, 64)
    a2 = jax.lax.dot_general(a1, w2_ref[...], (((1,), (0,)), ((), ())),
                             preferred_element_type=jnp.float32)
    a2 = jnp.maximum(a2, 0.0)
    out_ref[...] = jax.lax.dot_general(
        a2, w3_ref[...], (((1,), (0,)), ((), ())),
        preferred_element_type=jnp.float32)


def _mlp(enc3, w1tp, w2, w3):
    GB = 32
    grid = (G // GB,)
    return pl.pallas_call(
        _mlp_kernel,
        grid=grid,
        in_specs=[
            pl.BlockSpec((GB, HROWS, C), lambda i: (i, 0, 0)),
            pl.BlockSpec(w1tp.shape, lambda i: (0, 0)),
            pl.BlockSpec(w2.shape, lambda i: (0, 0)),
            pl.BlockSpec(w3.shape, lambda i: (0, 0)),
        ],
        out_specs=pl.BlockSpec((GB * C, 3), lambda i: (i, 0)),
        out_shape=jax.ShapeDtypeStruct((N_PTS, 3), jnp.float32),
    )(enc3, w1tp, w2, w3)


def kernel(x, table, W1, W2, W3):
    xp = x.reshape(G, C, 2).swapaxes(1, 2).reshape(2 * G, C)
    tables = [_pack_dense(table, l) if _DENSE[l]
              else table[l].reshape(T // 4, 8) for l in range(NUM_LEVELS)]
    enc3 = _sc_encode(xp, tables)
    w1tp = jnp.zeros((64, HROWS), jnp.float32).at[:, :2 + 2 * NUM_LEVELS].set(
        W1.T)
    return _mlp(enc3, w1tp, W2, W3)


# speedup vs baseline: 1.5223x; 1.5223x over previous
import dataclasses
import functools

import jax
import jax.numpy as jnp
import numpy as np
from jax import lax
from jax.experimental import pallas as pl
from jax.experimental.pallas import tpu as pltpu
from jax.experimental.pallas import tpu_sc as plsc

NUM_LEVELS = 16
F = 2
BASE_RES = 16
FINEST = 2048
T = 1 << 19
MASK = T - 1
PLS = float(np.exp2(np.log2(FINEST / BASE_RES) / (NUM_LEVELS - 1)))
P1_I32 = np.int32(np.uint32(2654435761).astype(np.int32))

N_PTS = 1048576
NC, NS = 2, 16
NW = NC * NS
PW = N_PTS // NW
C = 128
K = C // 128
NCHUNK = PW // C
G = N_PTS // C
HROWS = 40

_SCALES, _RES, _DENSE = [], [], []
for _l in range(NUM_LEVELS):
    _s = BASE_RES * (PLS ** _l) - 1.0
    _r = int(np.ceil(_s)) + 1
    _SCALES.append(np.float32(_s))
    _RES.append(_r)
    _DENSE.append(_r * _r <= T)
_DENSE_LVLS = [l for l in range(NUM_LEVELS) if _DENSE[l]]
_HASH_LVLS = [l for l in range(NUM_LEVELS) if not _DENSE[l]]
_DROWS = {l: _RES[l] * (_RES[l] + 1) + 2 for l in _DENSE_LVLS}

_SEG = {}
_seg = 0
for _l in range(NUM_LEVELS):
    _SEG[_l] = _seg
    _seg += K if _DENSE[_l] else 4 * K
NSEG = _seg
_NHSEG = 4 * K * len(_HASH_LVLS)


_NPAD = {l: ((_DROWS[l] + NW * 128 - 1) // (NW * 128)) * NW * 128
         for l in _DENSE_LVLS}
_QL = {l: _NPAD[l] // NW for l in _DENSE_LVLS}
_NB2 = {l: 2 * ((_QL[l] + _RES[l] + 2 + 127) // 128) for l in _DENSE_LVLS}
assert max(_NB2.values()) <= 176
_STG_ROWS = 176


def _sc_pack(tp):
    mesh = plsc.VectorSubcoreMesh(core_axis_name="c", subcore_axis_name="s")
    cp = pltpu.CompilerParams()
    if "needs_layout_passes" in pltpu.CompilerParams.__dataclass_fields__:
        cp = dataclasses.replace(cp, needs_layout_passes=False)
    if "use_tc_tiling_on_sc" in pltpu.CompilerParams.__dataclass_fields__:
        cp = dataclasses.replace(cp, use_tc_tiling_on_sc=False)

    out_types = [
        jax.ShapeDtypeStruct((_NPAD[l], 8), jnp.float32) if _DENSE[l]
        else jax.ShapeDtypeStruct((T // 4, 8), jnp.float32)
        for l in range(NUM_LEVELS)]

    @functools.partial(
        pl.kernel,
        compiler_params=cp,
        out_type=tuple(out_types),
        mesh=mesh,
        scratch_types=[
            pltpu.VMEM((_STG_ROWS, 128), jnp.float32),
            pltpu.VMEM((2, 128, 8), jnp.float32),
            pltpu.SemaphoreType.DMA,
        ],
    )
    def pack_kernel(*refs):
        tp_hbm = refs[0]
        out_hbm = list(refs[1:1 + NUM_LEVELS])
        stg, outb, sem = refs[1 + NUM_LEVELS:]
        wid = lax.axis_index("s") * NC + lax.axis_index("c")
        io16 = lax.iota(jnp.int32, 16)
        z16 = jnp.zeros((16,), jnp.int32)
        wcol = [z16 + w for w in range(8)]

        def ring_wait(b):
            pltpu.make_async_copy(
                outb.at[b], out_hbm[0].at[pl.ds(0, 128)], sem).wait()

        def dense_chunk(l, res, ob, oc, obase):
            @pl.loop(0, 128, step=16)
            def _(ii):
                oi = io16 + ii
                for p, off in enumerate((0, 1, res, res + 1)):
                    j = oi + (oc * 128 + off)
                    hi2 = lax.shift_right_logical(j, 7) * 2
                    lo = j & 127
                    v0 = plsc.load_gather(stg, [hi2, lo])
                    v1 = plsc.load_gather(stg, [hi2 + 1, lo])
                    plsc.store_scatter(ob, [oi, wcol[2 * p]], v0)
                    plsc.store_scatter(ob, [oi, wcol[2 * p + 1]], v1)

        def hash_chunk(l, ob, oc):
            @pl.loop(0, 128, step=16)
            def _(ii):
                oi = io16 + ii
                q4 = (oi + oc * 128) * 4
                for w in range(8):
                    f = w & 1
                    d = w >> 1
                    j = q4 + d
                    hi2 = lax.shift_right_logical(j, 7) * 2 + f
                    lo = j & 127
                    v = plsc.load_gather(stg, [hi2, lo])
                    plsc.store_scatter(ob, [oi, wcol[w]], v)

        def produce(chunks, fill, dst):
            pairs = chunks // 2
            if pairs:
                @pl.loop(0, 2 * pairs, step=2)
                def _(oc0):
                    for b in (0, 1):
                        oc = oc0 + b

                        @pl.when(oc0 > 0)
                        def _w(b=b):
                            ring_wait(b)
                        fill(outb.at[b], oc)
                        pltpu.async_copy(outb.at[b], dst(oc), sem)
                for b in (0, 1):
                    ring_wait(b)
            if chunks % 2:
                oc = chunks - 1
                fill(outb.at[0], oc)
                pltpu.async_copy(outb.at[0], dst(oc), sem)
                ring_wait(0)

        for l in range(NUM_LEVELS):
            if _DENSE[l]:
                res = _RES[l]
                q = _QL[l]
                nb2 = _NB2[l]
                pltpu.sync_copy(
                    tp_hbm.at[l].at[pl.ds(wid * (2 * q // 128), nb2)],
                    stg.at[pl.ds(0, nb2)])
                produce(
                    q // 128,
                    lambda ob, oc, l=l, res=res: dense_chunk(l, res, ob, oc, 0),
                    lambda oc, l=l, q=q: out_hbm[l].at[
                        pl.ds(wid * q + oc * 128, 128)])
            else:
                for ss in range(2):
                    pltpu.sync_copy(
                        tp_hbm.at[l].at[pl.ds(wid * 256 + ss * 128, 128)],
                        stg.at[pl.ds(0, 128)])
                    produce(
                        16,
                        lambda ob, oc, l=l: hash_chunk(l, ob, oc),
                        lambda oc, l=l, ss=ss: out_hbm[l].at[
                            pl.ds(wid * 4096 + ss * 2048 + oc * 128, 128)])

    return pack_kernel(tp)


def _sc_encode(xp, tables):
    mesh = plsc.VectorSubcoreMesh(core_axis_name="c", subcore_axis_name="s")
    cp = pltpu.CompilerParams()
    if "needs_layout_passes" in pltpu.CompilerParams.__dataclass_fields__:
        cp = dataclasses.replace(cp, needs_layout_passes=False)
    if "use_tc_tiling_on_sc" in pltpu.CompilerParams.__dataclass_fields__:
        cp = dataclasses.replace(cp, use_tc_tiling_on_sc=False)

    @functools.partial(
        pl.kernel,
        compiler_params=cp,
        out_type=jax.ShapeDtypeStruct((G, HROWS, C), jnp.float32),
        mesh=mesh,
        scratch_types=[
            pltpu.VMEM((C,), jnp.float32),
            pltpu.VMEM((C,), jnp.float32),
            pltpu.VMEM((NUM_LEVELS, C), jnp.float32),
            pltpu.VMEM((NUM_LEVELS, C), jnp.float32),
            pltpu.VMEM((NSEG, 128), jnp.int32),
            pltpu.VMEM((_NHSEG, 128), jnp.int32),
            pltpu.VMEM((NSEG * 128, 8), jnp.float32),
            pltpu.VMEM((HROWS, C), jnp.float32),
            pltpu.SemaphoreType.DMA,
            pltpu.SemaphoreType.DMA,
        ],
    )
    def enc_kernel(*refs):
        xp_hbm = refs[0]
        tbl_hbm = list(refs[1:1 + NUM_LEVELS])
        enc_hbm = refs[1 + NUM_LEVELS]
        (xv0, xv1, sfx, sfy, idxv, idxw, rowsv, outv,
         sem_g, sem_o) = refs[2 + NUM_LEVELS:]
        wid = lax.axis_index("s") * NC + lax.axis_index("c")
        io16 = lax.iota(jnp.int32, 16)

        @pl.loop(0, C, step=16)
        def _zpad(i):
            zf = jnp.zeros((16,), jnp.float32)
            for r in range(2 + 2 * NUM_LEVELS, HROWS):
                outv[r, pl.ds(i, 16)] = zf

        @pl.loop(0, NCHUNK)
        def _chunk(chunk):
            g = wid * NCHUNK + chunk
            pltpu.sync_copy(xp_hbm.at[2 * g], xv0)
            pltpu.sync_copy(xp_hbm.at[2 * g + 1], xv1)

            copies = []
            for l in range(NUM_LEVELS):
                scale = _SCALES[l]
                res = _RES[l]
                s0 = _SEG[l]
                for k in range(K):
                    @pl.loop(0, 128, step=16)
                    def _ixb(j, l=l, scale=scale, res=res, k=k, s0=s0):
                        i = k * 128 + j
                        vx = xv0[pl.ds(i, 16)]
                        vy = xv1[pl.ds(i, 16)]
                        px = vx * scale + 0.5
                        py = vy * scale + 0.5
                        pix = px.astype(jnp.int32)
                        piy = py.astype(jnp.int32)
                        fx = px - pix.astype(jnp.float32)
                        fy = py - piy.astype(jnp.float32)
                        sfx[l, pl.ds(i, 16)] = fx * fx * (3.0 - 2.0 * fx)
                        sfy[l, pl.ds(i, 16)] = fy * fy * (3.0 - 2.0 * fy)
                        if _DENSE[l]:
                            idxv[s0 + k, pl.ds(j, 16)] = piy * res + pix
                        else:
                            hy0 = piy * P1_I32
                            hy1 = hy0 + P1_I32
                            px1 = pix + 1
                            h0 = (_SEG[l] - _SEG[_HASH_LVLS[0]])
                            for c, gg in enumerate((
                                    (pix ^ hy0) & MASK, (px1 ^ hy0) & MASK,
                                    (pix ^ hy1) & MASK, (px1 ^ hy1) & MASK)):
                                idxv[s0 + c * K + k, pl.ds(j, 16)] = (
                                    lax.shift_right_logical(gg, 2))
                                idxw[h0 + c * K + k, pl.ds(j, 16)] = (
                                    (gg & 3) * 2)

                nseg_l = K if _DENSE[l] else 4 * K
                for s in range(s0, s0 + nseg_l):
                    copies.append(pltpu.async_copy(
                        tbl_hbm[l].at[idxv.at[s]],
                        rowsv.at[pl.ds(s * 128, 128)],
                        sem_g))

            for cp_ in copies:
                cp_.wait()

            @pl.when(chunk > 0)
            def _():
                pltpu.make_async_copy(
                    outv, enc_hbm.at[g], sem_o).wait()

            @pl.loop(0, C, step=16)
            def _xrow(i):
                outv[0, pl.ds(i, 16)] = 2.0 * xv0[pl.ds(i, 16)] - 1.0
                outv[1, pl.ds(i, 16)] = 2.0 * xv1[pl.ds(i, 16)] - 1.0

            z16 = jnp.zeros((16,), jnp.int32)
            wcol = [z16 + w for w in range(8)]
            for l in range(NUM_LEVELS):
                s0 = _SEG[l]

                @pl.loop(0, C, step=16)
                def _cmb(i, l=l, s0=s0):
                    sx = sfx[l, pl.ds(i, 16)]
                    sy = sfy[l, pl.ds(i, 16)]
                    wx0 = 1.0 - sx
                    wy0 = 1.0 - sy
                    w00 = wx0 * wy0
                    w10 = sx * wy0
                    w01 = wx0 * sy
                    w11 = sx * sy
                    pi = io16 + i
                    if _DENSE[l]:
                        row = pi + s0 * 128
                        gv = [plsc.load_gather(rowsv, [row, wcol[w]])
                              for w in range(8)]
                        a0 = (w00 * gv[0] + w10 * gv[2]
                              + w01 * gv[4] + w11 * gv[6])
                        a1 = (w00 * gv[1] + w10 * gv[3]
                              + w01 * gv[5] + w11 * gv[7])
                    else:
                        h0 = s0 - _SEG[_HASH_LVLS[0]]
                        ws = [w00, w10, w01, w11]
                        a0 = None
                        a1 = None
                        for c in range(4):
                            r = pi + (s0 + c * K) * 128
                            wb = idxw[h0 + c * K, pl.ds(i, 16)]
                            f0 = plsc.load_gather(rowsv, [r, wb])
                            f1 = plsc.load_gather(rowsv, [r, wb + 1])
                            a0 = ws[c] * f0 if a0 is None else a0 + ws[c] * f0
                            a1 = ws[c] * f1 if a1 is None else a1 + ws[c] * f1
                    outv[2 + 2 * l, pl.ds(i, 16)] = a0
                    outv[3 + 2 * l, pl.ds(i, 16)] = a1

            pltpu.async_copy(outv, enc_hbm.at[g], sem_o)

        pltpu.make_async_copy(outv, enc_hbm.at[0], sem_o).wait()

    return enc_kernel(xp, *tables)


def _mlp_kernel(h_ref, w1tp_ref, w2_ref, w3_ref, out_ref):
    h3 = h_ref[...]
    gb = h3.shape[0]
    a1 = jax.lax.dot_general(h3, w1tp_ref[...], (((1,), (1,)), ((), ())),
                             preferred_element_type=jnp.float32)
    a1 = jnp.maximum(a1, 0.0).reshape(gb * C, 64)
    a2 = jax.lax.dot_general(a1, w2_ref[...], (((1,), (0,)), ((), ())),
                             preferred_element_type=jnp.float32)
    a2 = jnp.maximum(a2, 0.0)
    out_ref[...] = jax.lax.dot_general(
        a2, w3_ref[...], (((1,), (0,)), ((), ())),
        preferred_element_type=jnp.float32)


def _mlp(enc3, w1tp, w2, w3):
    GB = 32
    grid = (G // GB,)
    return pl.pallas_call(
        _mlp_kernel,
        grid=grid,
        in_specs=[
            pl.BlockSpec((GB, HROWS, C), lambda i: (i, 0, 0)),
            pl.BlockSpec(w1tp.shape, lambda i: (0, 0)),
            pl.BlockSpec(w2.shape, lambda i: (0, 0)),
            pl.BlockSpec(w3.shape, lambda i: (0, 0)),
        ],
        out_specs=pl.BlockSpec((GB * C, 3), lambda i: (i, 0)),
        out_shape=jax.ShapeDtypeStruct((N_PTS, 3), jnp.float32),
    )(enc3, w1tp, w2, w3)


def kernel(x, table, W1, W2, W3):
    xp = x.reshape(G, C, 2).swapaxes(1, 2).reshape(2 * G, C)
    tp = table.reshape(NUM_LEVELS, 4096, 128, 2).swapaxes(2, 3).reshape(
        NUM_LEVELS, 8192, 128)
    tables = _sc_pack(tp)
    enc3 = _sc_encode(xp, tables)
    w1tp = jnp.zeros((64, HROWS), jnp.float32).at[:, :2 + 2 * NUM_LEVELS].set(
        W1.T)
    return _mlp(enc3, w1tp, W2, W3)
